# baseline (speedup 1.0000x reference)
import functools
import math

import jax
import jax.numpy as jnp
from jax import lax
from jax.experimental import pallas as pl
from jax.experimental.pallas import tpu as pltpu
from jax.experimental.pallas import tpu_sc as plsc

DIM = 64
LANES = 16

_NC = 2
_NS = 16
_NW = _NC * _NS

_CHUNK = 128
_BATCH = 64
_PE_W = 216


def _rsqrt_vec(x):
    i = plsc.bitcast(x, jnp.int32)
    i = jnp.full((LANES,), 0x5F3759DF, jnp.int32) - lax.shift_right_logical(i, 1)
    r = plsc.bitcast(i, jnp.float32)
    hx = 0.5 * x
    for _ in range(3):
        r = r * (1.5 - hx * r * r)
    return r


def _fused_embed_ln(table, idx, pe_t, gamma, beta, n_rows, seq_len):
    per_w = n_rows // _NW
    n_chunks = per_w // _CHUNK

    mesh = plsc.VectorSubcoreMesh(
        core_axis_name="c", subcore_axis_name="s",
        num_cores=_NC, num_subcores=_NS)

    @functools.partial(
        pl.kernel,
        mesh=mesh,
        out_type=jax.ShapeDtypeStruct((n_rows, DIM), jnp.float32),
        scratch_types=[
            pltpu.VMEM((2, _CHUNK), jnp.int32),
            pltpu.VMEM((2, _CHUNK, 2 * DIM), jnp.float32),
            pltpu.VMEM((2, _CHUNK, DIM), jnp.float32),
            pltpu.VMEM((_CHUNK,), jnp.float32),
            pltpu.VMEM((_CHUNK,), jnp.float32),
            pltpu.VMEM(pe_t.shape, jnp.float32),
            pltpu.VMEM((DIM,), jnp.float32),
            pltpu.VMEM((DIM,), jnp.float32),
            pltpu.SemaphoreType.DMA,
            pltpu.SemaphoreType.DMA,
        ],
        compiler_params=pltpu.CompilerParams(
            use_tc_tiling_on_sc=False, needs_layout_passes=False),
    )
    def k(table_hbm, idx_hbm, pe_hbm, g_hbm, b_hbm, out_hbm,
          idx_v, rows_v, stage_v, mu_v, rs_v, pe_v, g_v, b_v, sem_g, sem_o):
        wid = lax.axis_index("s") * _NC + lax.axis_index("c")
        wbase = wid * per_w

        pltpu.sync_copy(pe_hbm, pe_v)
        pltpu.sync_copy(g_hbm, g_v)
        pltpu.sync_copy(b_hbm, b_v)

        g_regs = [g_v[pl.ds(LANES * t, LANES)] for t in range(DIM // LANES)]
        b_regs = [b_v[pl.ds(LANES * t, LANES)] for t in range(DIM // LANES)]
        lane_iota = lax.iota(jnp.int32, LANES)

        def fire_gather(c):
            p = lax.rem(c, 2)
            base = wbase + c * _CHUNK
            pltpu.sync_copy(idx_hbm.at[pl.ds(base, _CHUNK)], idx_v.at[p])
            pltpu.async_copy(
                table_hbm.at[idx_v.at[p]], rows_v.at[p], sem_g)

        def wait_gather(c):
            p = lax.rem(c, 2)
            pltpu.make_async_copy(
                table_hbm.at[idx_v.at[p]], rows_v.at[p], sem_g).wait()

        def compute(c):
            p = lax.rem(c, 2)
            base = wbase + c * _CHUNK

            def stats_body(gi, _):
                l0 = lax.rem(base + gi * LANES, seq_len)
                rows16 = gi * LANES + lane_iota
                acc_s = jnp.zeros((LANES,), jnp.float32)
                acc_q = jnp.zeros((LANES,), jnp.float32)
                for d in range(DIM):
                    v = plsc.load_gather(
                        rows_v.at[p], [rows16, jnp.full((LANES,), d, jnp.int32)])
                    v = v + pe_v[pl.ds(d * _PE_W + l0, LANES)]
                    acc_s = acc_s + v
                    acc_q = acc_q + v * v
                mu = acc_s * (1.0 / DIM)
                var = acc_q * (1.0 / DIM) - mu * mu
                mu_v[pl.ds(gi * LANES, LANES)] = mu
                rs_v[pl.ds(gi * LANES, LANES)] = _rsqrt_vec(var + 1e-5)
                return ()

            lax.fori_loop(0, _CHUNK // LANES, stats_body, (), unroll=False)

            def norm_body(gi, _):
                muw = mu_v[pl.ds(gi * LANES, LANES)]
                rsw = rs_v[pl.ds(gi * LANES, LANES)]
                for t in range(LANES):
                    r = gi * LANES + t
                    l = lax.rem(base + r, seq_len)
                    mu = muw[t]
                    rstd = rsw[t]
                    for u in range(DIM // LANES):
                        pe16 = plsc.load_gather(
                            pe_v, [(u * LANES + lane_iota) * _PE_W
                                   + jnp.full((LANES,), l, jnp.int32)])
                        e = rows_v[p, r, pl.ds(LANES * u, LANES)] + pe16
                        stage_v[p, r, pl.ds(LANES * u, LANES)] = (
                            (e - mu) * rstd * g_regs[u] + b_regs[u])
                return ()

            lax.fori_loop(0, _CHUNK // LANES, norm_body, (), unroll=False)
            pltpu.async_copy(
                stage_v.at[p], out_hbm.at[pl.ds(base, _CHUNK)], sem_o)

        def drain_out():
            pltpu.make_async_copy(
                stage_v.at[0], out_hbm.at[pl.ds(wbase, _CHUNK)],
                sem_o).wait()

        def chunk_body(c, _):
            @pl.when(c + 1 < n_chunks)
            def _():
                fire_gather(c + 1)

            wait_gather(c)

            @pl.when(c >= 2)
            def _():
                drain_out()

            compute(c)
            return ()

        fire_gather(0)
        lax.fori_loop(0, n_chunks, chunk_body, (), unroll=False)
        drain_out()
        drain_out()

    return k(table, idx, pe_t, gamma, beta)


def _pe_table(length, d):
    position = jnp.arange(length, dtype=jnp.float32)[:, None]
    div_term = jnp.exp(
        jnp.arange(0, d, 2, dtype=jnp.float32) * (-math.log(10000.0) / d))
    ang = position * div_term
    return jnp.stack([jnp.sin(ang), jnp.cos(ang)], axis=-1).reshape(length, d)


def kernel(x, word_embeddings_weight, ln_gamma, ln_beta):
    b, l = x.shape
    n = b * l
    pe = _pe_table(l, DIM)
    pe_t = jnp.concatenate([pe.T, pe.T[:, :LANES]], axis=1).reshape(-1)
    table128 = jnp.pad(word_embeddings_weight, ((0, 0), (0, DIM)))
    out = _fused_embed_ln(table128, x.reshape(n), pe_t,
                          ln_gamma, ln_beta, n, l)
    return out.reshape(b, l, DIM)


# speedup vs baseline: 1.5094x; 1.5094x over previous
import functools
import math

import jax
import jax.numpy as jnp
from jax import lax
from jax.experimental import pallas as pl
from jax.experimental.pallas import tpu as pltpu
from jax.experimental.pallas import tpu_sc as plsc

DIM = 64
ROW_W = 128

_NC = 2
_NS = 16
_NW = _NC * _NS

_CHUNK = 256
_IDX_W = 128


def _sc_gather(table128, idx, n_rows):
    per_w = n_rows // _NW
    n_chunks = per_w // _CHUNK
    n_streams = _CHUNK // _IDX_W

    mesh = plsc.VectorSubcoreMesh(
        core_axis_name="c", subcore_axis_name="s",
        num_cores=_NC, num_subcores=_NS)

    @functools.partial(
        pl.kernel,
        mesh=mesh,
        out_type=jax.ShapeDtypeStruct((n_rows, ROW_W), jnp.float32),
        scratch_types=[
            pltpu.VMEM((2, _CHUNK), jnp.int32),
            pltpu.VMEM((2, _CHUNK, ROW_W), jnp.float32),
            pltpu.SemaphoreType.DMA,
            pltpu.SemaphoreType.DMA,
        ],
        compiler_params=pltpu.CompilerParams(use_tc_tiling_on_sc=False),
    )
    def k(table_hbm, idx_hbm, out_hbm, idx_v, rows_v, sem_g, sem_o):
        wid = lax.axis_index("s") * _NC + lax.axis_index("c")
        wbase = wid * per_w

        def fire_gather(c):
            p = lax.rem(c, 2)
            base = wbase + c * _CHUNK
            pltpu.sync_copy(idx_hbm.at[pl.ds(base, _CHUNK)], idx_v.at[p])
            for j in range(n_streams):
                pltpu.async_copy(
                    table_hbm.at[idx_v.at[p, pl.ds(j * _IDX_W, _IDX_W)]],
                    rows_v.at[p, pl.ds(j * _IDX_W, _IDX_W)],
                    sem_g)

        def wait_gather(c):
            p = lax.rem(c, 2)
            for j in range(n_streams):
                pltpu.make_async_copy(
                    table_hbm.at[idx_v.at[p, pl.ds(j * _IDX_W, _IDX_W)]],
                    rows_v.at[p, pl.ds(j * _IDX_W, _IDX_W)],
                    sem_g).wait()

        def drain_out():
            pltpu.make_async_copy(
                rows_v.at[0], out_hbm.at[pl.ds(wbase, _CHUNK)],
                sem_o).wait()

        def chunk_body(c, _):
            @pl.when(c >= 1)
            def _():
                drain_out()

            @pl.when(c + 1 < n_chunks)
            def _():
                fire_gather(c + 1)

            wait_gather(c)

            p = lax.rem(c, 2)
            base = wbase + c * _CHUNK
            pltpu.async_copy(
                rows_v.at[p], out_hbm.at[pl.ds(base, _CHUNK)], sem_o)
            return ()

        fire_gather(0)
        lax.fori_loop(0, n_chunks, chunk_body, (), unroll=False)
        drain_out()

    return k(table128, idx)


def _ln_body(emb_ref, pe_ref, g_ref, b_ref, out_ref):
    e = emb_ref[..., :DIM] + pe_ref[...]
    mu = jnp.mean(e, axis=-1, keepdims=True)
    var = jnp.mean(jnp.square(e - mu), axis=-1, keepdims=True)
    out_ref[...] = (e - mu) * lax.rsqrt(var + 1e-5) * g_ref[...] + b_ref[...]


def _tc_ln(emb, pe, gamma, beta):
    b, l, d2 = emb.shape
    d = DIM
    bb = 16
    return pl.pallas_call(
        _ln_body,
        grid=(b // bb,),
        in_specs=[
            pl.BlockSpec((bb, l, d2), lambda i: (i, 0, 0)),
            pl.BlockSpec((1, l, d), lambda i: (0, 0, 0)),
            pl.BlockSpec((1, 1, d), lambda i: (0, 0, 0)),
            pl.BlockSpec((1, 1, d), lambda i: (0, 0, 0)),
        ],
        out_specs=pl.BlockSpec((bb, l, d), lambda i: (i, 0, 0)),
        out_shape=jax.ShapeDtypeStruct((b, l, d), jnp.float32),
    )(emb, pe, gamma, beta)


def _pe_table(length, d):
    position = jnp.arange(length, dtype=jnp.float32)[:, None]
    div_term = jnp.exp(
        jnp.arange(0, d, 2, dtype=jnp.float32) * (-math.log(10000.0) / d))
    ang = position * div_term
    return jnp.stack([jnp.sin(ang), jnp.cos(ang)], axis=-1).reshape(length, d)


def kernel(x, word_embeddings_weight, ln_gamma, ln_beta):
    b, l = x.shape
    n = b * l
    table128 = jnp.pad(word_embeddings_weight, ((0, 0), (0, ROW_W - DIM)))
    gathered = _sc_gather(table128, x.reshape(n), n)
    pe = _pe_table(l, DIM)[None]
    g = ln_gamma.reshape(1, 1, DIM)
    be = ln_beta.reshape(1, 1, DIM)
    return _tc_ln(gathered.reshape(b, l, ROW_W), pe, g, be)


# speedup vs baseline: 1.5385x; 1.0193x over previous
import functools
import math

import jax
import jax.numpy as jnp
from jax import lax
from jax.experimental import pallas as pl
from jax.experimental.pallas import tpu as pltpu
from jax.experimental.pallas import tpu_sc as plsc

DIM = 64
ROW_W = 128

_NC = 2
_NS = 16
_NW = _NC * _NS

_CHUNK = 256
_IDX_W = 128


def _sc_gather(table128, idx, n_rows):
    per_w = n_rows // _NW
    n_chunks = per_w // _CHUNK
    n_streams = _CHUNK // _IDX_W

    mesh = plsc.VectorSubcoreMesh(
        core_axis_name="c", subcore_axis_name="s",
        num_cores=_NC, num_subcores=_NS)

    @functools.partial(
        pl.kernel,
        mesh=mesh,
        out_type=jax.ShapeDtypeStruct((n_rows, ROW_W), jnp.float32),
        scratch_types=[
            pltpu.VMEM((2, _CHUNK), jnp.int32),
            pltpu.VMEM((2, _CHUNK, ROW_W), jnp.float32),
            pltpu.SemaphoreType.DMA,
            pltpu.SemaphoreType.DMA,
        ],
        compiler_params=pltpu.CompilerParams(use_tc_tiling_on_sc=False),
    )
    def k(table_hbm, idx_hbm, out_hbm, idx_v, rows_v, sem_g, sem_o):
        wid = lax.axis_index("s") * _NC + lax.axis_index("c")
        wbase = wid * per_w

        def fire_gather(c):
            p = lax.rem(c, 2)
            base = wbase + c * _CHUNK
            pltpu.sync_copy(idx_hbm.at[pl.ds(base, _CHUNK)], idx_v.at[p])
            for j in range(n_streams):
                pltpu.async_copy(
                    table_hbm.at[idx_v.at[p, pl.ds(j * _IDX_W, _IDX_W)]],
                    rows_v.at[p, pl.ds(j * _IDX_W, _IDX_W)],
                    sem_g)

        def wait_gather(c):
            p = lax.rem(c, 2)
            for j in range(n_streams):
                pltpu.make_async_copy(
                    table_hbm.at[idx_v.at[p, pl.ds(j * _IDX_W, _IDX_W)]],
                    rows_v.at[p, pl.ds(j * _IDX_W, _IDX_W)],
                    sem_g).wait()

        def drain_out():
            pltpu.make_async_copy(
                rows_v.at[0], out_hbm.at[pl.ds(wbase, _CHUNK)],
                sem_o).wait()

        def chunk_body(c, _):
            @pl.when(c >= 1)
            def _():
                drain_out()

            @pl.when(c + 1 < n_chunks)
            def _():
                fire_gather(c + 1)

            wait_gather(c)

            p = lax.rem(c, 2)
            base = wbase + c * _CHUNK
            pltpu.async_copy(
                rows_v.at[p], out_hbm.at[pl.ds(base, _CHUNK)], sem_o)
            return ()

        fire_gather(0)
        lax.fori_loop(0, n_chunks, chunk_body, (), unroll=False)
        drain_out()

    return k(table128, idx)


def _ln_body(emb_ref, pe_ref, g_ref, b_ref, out_ref):
    e = emb_ref[..., :DIM] + pe_ref[...]
    mu = jnp.mean(e, axis=-1, keepdims=True)
    var = jnp.mean(jnp.square(e - mu), axis=-1, keepdims=True)
    out_ref[...] = (e - mu) * lax.rsqrt(var + 1e-5) * g_ref[...] + b_ref[...]


def _tc_ln(emb, pe, gamma, beta):
    b, l, d2 = emb.shape
    d = DIM
    bb = 32
    return pl.pallas_call(
        _ln_body,
        grid=(b // bb,),
        in_specs=[
            pl.BlockSpec((bb, l, d2), lambda i: (i, 0, 0)),
            pl.BlockSpec((1, l, d), lambda i: (0, 0, 0)),
            pl.BlockSpec((1, 1, d), lambda i: (0, 0, 0)),
            pl.BlockSpec((1, 1, d), lambda i: (0, 0, 0)),
        ],
        out_specs=pl.BlockSpec((bb, l, d), lambda i: (i, 0, 0)),
        out_shape=jax.ShapeDtypeStruct((b, l, d), jnp.float32),
    )(emb, pe, gamma, beta)


def _pe_table(length, d):
    position = jnp.arange(length, dtype=jnp.float32)[:, None]
    div_term = jnp.exp(
        jnp.arange(0, d, 2, dtype=jnp.float32) * (-math.log(10000.0) / d))
    ang = position * div_term
    return jnp.stack([jnp.sin(ang), jnp.cos(ang)], axis=-1).reshape(length, d)


def kernel(x, word_embeddings_weight, ln_gamma, ln_beta):
    b, l = x.shape
    n = b * l
    table128 = jnp.pad(word_embeddings_weight, ((0, 0), (0, ROW_W - DIM)))
    gathered = _sc_gather(table128, x.reshape(n), n)
    pe = _pe_table(l, DIM)[None]
    g = ln_gamma.reshape(1, 1, DIM)
    be = ln_beta.reshape(1, 1, DIM)
    return _tc_ln(gathered.reshape(b, l, ROW_W), pe, g, be)
